# Initial kernel scaffold; baseline (speedup 1.0000x reference)
import jax
import jax.numpy as jnp
from jax.experimental import pallas as pl


def kernel(x, W1a, b1a, W1b, b1b, Wla, bla, Wlb, blb, eps, Wm1, bm1, Wm2, bm2, Wih, Whh, bih, bhh, Wfc1, bfc1, Wfc4, bfc4, edge_index_1_l, edge_index_2_l, edge_index_1_g, edge_index_2_g, batch):
    raise NotImplementedError("write your pallas kernel here")


# speedup vs baseline: 1.5369x; 1.5369x over previous
import jax
import jax.numpy as jnp
from jax import lax
from jax.experimental import pallas as pl
from jax.experimental.pallas import tpu as pltpu
from jax.experimental.pallas import tpu_sc as plsc

_N = 50000
_D = 64
_H = 32
_G = 256
_STEPS = 6
_NC = 2
_NS = 16
_CHUNK = 128
_E = 800000
_CPS = 391
_E_PAD = _NS * _CHUNK * _CPS
_N_ACC = 51200
_ZROWS = _N_ACC // _NS


def _make_segsum(jobs):
    njobs = len(jobs)

    def body(tab_hbm, src_hbm, dst_hbm, zeros_hbm, out_hbm,
             srcb, idxb, dstb, rows, accum, sem):
        c = lax.axis_index("c")
        s = lax.axis_index("s")
        for j, (e, p) in enumerate(jobs):
            pltpu.sync_copy(zeros_hbm, accum.at[pl.ds(s * _ZROWS, _ZROWS)])
            plsc.subcore_barrier()
            off = (p * _NC + c) * _N

            def chunk(t, carry):
                base = s * (_CPS * _CHUNK) + t * _CHUNK
                pltpu.sync_copy(src_hbm.at[e, pl.ds(base, _CHUNK)], srcb)
                pltpu.sync_copy(dst_hbm.at[e, pl.ds(base, _CHUNK)], dstb)
                for k in range(_CHUNK // 16):
                    idxb[pl.ds(k * 16, 16)] = srcb[pl.ds(k * 16, 16)] + off
                pltpu.async_copy(tab_hbm.at[idxb], rows, sem).wait()
                pltpu.sync_copy(rows, accum.at[dstb], add=True)
                return carry

            lax.fori_loop(0, _CPS, chunk, 0)
            plsc.subcore_barrier()
            pltpu.sync_copy(accum.at[pl.ds(s * _ZROWS, _ZROWS)],
                            out_hbm.at[j, c, pl.ds(s * _ZROWS, _ZROWS)])
            plsc.subcore_barrier()

    return pl.kernel(
        body,
        out_type=jax.ShapeDtypeStruct((njobs, _NC, _N_ACC, _H), jnp.float32),
        mesh=plsc.VectorSubcoreMesh(core_axis_name="c", subcore_axis_name="s",
                                    num_cores=_NC, num_subcores=_NS),
        scratch_types=[
            pltpu.VMEM((_CHUNK,), jnp.int32),
            pltpu.VMEM((_CHUNK,), jnp.int32),
            pltpu.VMEM((_CHUNK,), jnp.int32),
            pltpu.VMEM((_CHUNK, _H), jnp.float32),
            pltpu.VMEM_SHARED((_N_ACC, _H), jnp.float32),
            pltpu.SemaphoreType.DMA,
        ],
        compiler_params=pltpu.CompilerParams(use_tc_tiling_on_sc=False),
    )


_segsum_l0 = _make_segsum([(e, p) for e in range(4) for p in range(2)])
_segsum_ln = _make_segsum([(e, 0) for e in range(4)])


def _agg4_l0(x, src4, dst4, zeros):
    x128 = jnp.pad(x, ((0, 0), (0, 128 - x.shape[1])))
    tab = x128.reshape(_N, 4, _H).transpose(1, 0, 2).reshape(4 * _N, _H)
    out = _segsum_l0(tab, src4, dst4, zeros)
    out = out[:, :, :_N, :].reshape(4, 4, _N, _H)
    return out.transpose(0, 2, 1, 3).reshape(4, _N, 128)[:, :, :x.shape[1]]


def _agg4_ln(x, src4, dst4, zeros):
    tab = x.reshape(_N, 2, _H).transpose(1, 0, 2).reshape(2 * _N, _H)
    out = _segsum_ln(tab, src4, dst4, zeros)
    return out[:, :, :_N, :].transpose(0, 2, 1, 3).reshape(4, _N, _D)


def _bn(h):
    m = jnp.mean(h, axis=0)
    v = jnp.mean((h - m) ** 2, axis=0)
    return (h - m) / jnp.sqrt(v + 1e-5)


def _layer(xl, Wa4, ba4, Wb4, bb4, eps4, Wm1l, bm1l, Wm2l, bm2l, agg4):
    hs = []
    for i in range(4):
        h = (1.0 + eps4[i]) * xl + agg4[i]
        h = jax.nn.relu(_bn(h @ Wa4[i].T + ba4[i]))
        h = jax.nn.relu(_bn(h @ Wb4[i].T + bb4[i]))
        hs.append(h)
    cat = jnp.concatenate([hs[0], hs[2], hs[1], hs[3]], axis=-1)
    y = jax.nn.relu(_bn(cat @ Wm1l.T + bm1l))
    return jax.nn.relu(_bn(y @ Wm2l.T + bm2l))


def _pool(x, batch, Wih, Whh, bih, bhh):
    q_star = jnp.zeros((_G, 2 * _D), dtype=x.dtype)
    h = jnp.zeros((_G, _D), dtype=x.dtype)
    c = jnp.zeros((_G, _D), dtype=x.dtype)
    for _ in range(_STEPS):
        gates = q_star @ Wih.T + bih + h @ Whh.T + bhh
        i, f, g, o = jnp.split(gates, 4, axis=-1)
        i = jax.nn.sigmoid(i); f = jax.nn.sigmoid(f)
        g = jnp.tanh(g); o = jax.nn.sigmoid(o)
        c = f * c + i * g
        h = o * jnp.tanh(c)
        q = h
        e = jnp.sum(x * q[batch], axis=-1)
        emax = jax.ops.segment_max(e, batch, num_segments=_G)
        emax = jnp.where(jnp.isfinite(emax), emax, 0.0)
        a = jnp.exp(e - emax[batch])
        asum = jax.ops.segment_sum(a, batch, num_segments=_G)
        a = a / (asum[batch] + 1e-16)
        r = jax.ops.segment_sum(a[:, None] * x, batch, num_segments=_G)
        q_star = jnp.concatenate([q, r], axis=-1)
    return q_star


def _prep_edges(edges):
    pad = _E_PAD - _E
    ar = jnp.arange(pad, dtype=jnp.int32)
    pad_src = ar % _N
    pad_dst = _N + (ar % (_N_ACC - _N))
    srcs, dsts = [], []
    for ei in edges:
        src = ei[0].astype(jnp.int32)
        dst = ei[1].astype(jnp.int32)
        perm = jnp.argsort(dst, stable=True)
        srcs.append(jnp.concatenate([src[perm], pad_src]))
        dsts.append(jnp.concatenate([dst[perm], pad_dst]))
    return jnp.stack(srcs), jnp.stack(dsts)


def kernel(x, W1a, b1a, W1b, b1b, Wla, bla, Wlb, blb, eps, Wm1, bm1, Wm2, bm2,
           Wih, Whh, bih, bhh, Wfc1, bfc1, Wfc4, bfc4,
           edge_index_1_l, edge_index_2_l, edge_index_1_g, edge_index_2_g,
           batch):
    edges = [edge_index_1_l, edge_index_2_l, edge_index_1_g, edge_index_2_g]
    src4, dst4 = _prep_edges(edges)
    zeros = jnp.zeros((_ZROWS, _H), dtype=jnp.float32)

    agg = _agg4_l0(x, src4, dst4, zeros)
    xr = _layer(x, W1a, b1a, W1b, b1b, eps[0], Wm1[0], bm1[0], Wm2[0], bm2[0],
                agg)
    for l in range(1, 6):
        agg = _agg4_ln(xr, src4, dst4, zeros)
        xr = _layer(xr, Wla[l - 1], bla[l - 1], Wlb[l - 1], blb[l - 1],
                    eps[l], Wm1[l], bm1[l], Wm2[l], bm2[l], agg)
    q_star = _pool(xr, batch, Wih, Whh, bih, bhh)
    out = jax.nn.relu(q_star @ Wfc1.T + bfc1) @ Wfc4.T + bfc4
    return out
